# Initial kernel scaffold; baseline (speedup 1.0000x reference)
import jax
import jax.numpy as jnp
from jax.experimental import pallas as pl


def kernel(x, edge_index, W1_l, b1_l, W1_r, W2_l, b2_l, W2_r):
    raise NotImplementedError("write your pallas kernel here")


# speedup vs baseline: 4.6585x; 4.6585x over previous
import functools

import jax
import jax.numpy as jnp
from jax import lax
from jax.experimental import pallas as pl
from jax.experimental.pallas import tpu as pltpu
from jax.experimental.pallas import tpu_sc as plsc

N = 10000
D = 256
H = 128
E = 160000
K = 128
NCHUNK = E // K
NSUB = 16
NITER = -(-NCHUNK // NSUB)
ROWS_A = 624
ROWS_TAIL = N - NSUB * ROWS_A
CNT_TILES = 10
CNT_ROWS = N // CNT_TILES

RB = 1000
GRID = N // RB


def _agg_body(compute_counts, *refs):
    if compute_counts:
        (x_lo, x_hi, src, dst, zrows, zvec,
         out_lo, out_hi, out_cnt,
         src_v, dst_v, rows_v, ones_v, cz_v, accum, cnt_acc, sem) = refs
    else:
        (x_lo, x_hi, src, dst, zrows,
         out_lo, out_hi,
         src_v, dst_v, rows_v, accum, sem) = refs

    c = lax.axis_index("c")
    s = lax.axis_index("s")
    r0 = s * ROWS_A

    def _rows_copy(src_ref, dst_ref):
        pltpu.sync_copy(src_ref.at[pl.ds(r0, ROWS_A)],
                        dst_ref.at[pl.ds(r0, ROWS_A)])

        @pl.when(s == 0)
        def _():
            pltpu.sync_copy(src_ref.at[pl.ds(NSUB * ROWS_A, ROWS_TAIL)],
                            dst_ref.at[pl.ds(NSUB * ROWS_A, ROWS_TAIL)])

    _rows_copy(zrows, accum)
    if compute_counts:
        @pl.when(jnp.logical_and(c == 0, s < CNT_TILES))
        def _():
            q0 = s * CNT_ROWS
            pltpu.sync_copy(zvec.at[pl.ds(q0, CNT_ROWS)], cz_v)
            pltpu.sync_copy(cz_v, cnt_acc.at[pl.ds(q0, CNT_ROWS)])
        for k in range(K // 16):
            ones_v[pl.ds(k * 16, 16)] = jnp.full((16,), 1.0, jnp.float32)
    plsc.subcore_barrier()

    def step(i, carry):
        j = i * NSUB + s

        @pl.when(j < NCHUNK)
        def _():
            e0 = j * K
            pltpu.sync_copy(src.at[pl.ds(e0, K)], src_v)
            pltpu.sync_copy(dst.at[pl.ds(e0, K)], dst_v)

            @pl.when(c == 0)
            def _():
                pltpu.async_copy(x_lo.at[src_v], rows_v, sem).wait()

            @pl.when(c == 1)
            def _():
                pltpu.async_copy(x_hi.at[src_v], rows_v, sem).wait()

            pltpu.sync_copy(rows_v, accum.at[dst_v], add=True)
            if compute_counts:
                @pl.when(c == 0)
                def _():
                    pltpu.sync_copy(ones_v, cnt_acc.at[dst_v], add=True)

        return carry

    lax.fori_loop(0, NITER, step, 0)
    plsc.subcore_barrier()

    @pl.when(c == 0)
    def _():
        _rows_copy(accum, out_lo)

    @pl.when(c == 1)
    def _():
        _rows_copy(accum, out_hi)

    if compute_counts:
        @pl.when(jnp.logical_and(c == 0, s < CNT_TILES))
        def _():
            q0 = s * CNT_ROWS
            pltpu.sync_copy(cnt_acc.at[pl.ds(q0, CNT_ROWS)], cz_v)
            pltpu.sync_copy(cz_v, out_cnt.at[pl.ds(q0, CNT_ROWS)])


def _make_agg(compute_counts):
    out_type = [jax.ShapeDtypeStruct((N, H), jnp.float32),
                jax.ShapeDtypeStruct((N, H), jnp.float32)]
    scratch = [
        pltpu.VMEM((K,), jnp.int32),
        pltpu.VMEM((K,), jnp.int32),
        pltpu.VMEM((K, H), jnp.float32),
    ]
    if compute_counts:
        out_type.append(jax.ShapeDtypeStruct((N,), jnp.float32))
        scratch.append(pltpu.VMEM((K,), jnp.float32))
        scratch.append(pltpu.VMEM((CNT_ROWS,), jnp.float32))
    scratch.append(pltpu.VMEM_SHARED((N, H), jnp.float32))
    if compute_counts:
        scratch.append(pltpu.VMEM_SHARED((N,), jnp.float32))
    scratch.append(pltpu.SemaphoreType.DMA)
    mesh = plsc.VectorSubcoreMesh(core_axis_name="c", subcore_axis_name="s")
    return pl.kernel(functools.partial(_agg_body, compute_counts),
                     out_type=tuple(out_type), mesh=mesh,
                     scratch_types=scratch)


_agg_cnt = _make_agg(True)
_agg_nocnt = _make_agg(False)


def _dense_body(relu, split, a_lo, a_hi, cnt, r_lo, r_hi, wl, wr, b, *outs):
    inv = 1.0 / jnp.maximum(cnt[...], 1.0)
    wlv = wl[...]
    wrv = wr[...]
    acc = jnp.dot(a_lo[...] * inv, wlv[:H], preferred_element_type=jnp.float32)
    acc += jnp.dot(a_hi[...] * inv, wlv[H:], preferred_element_type=jnp.float32)
    acc += jnp.dot(r_lo[...], wrv[:H], preferred_element_type=jnp.float32)
    acc += jnp.dot(r_hi[...], wrv[H:], preferred_element_type=jnp.float32)
    acc += b[...]
    if relu:
        acc = jnp.maximum(acc, 0.0)
    if split:
        outs[0][...] = acc[:, :H]
        outs[1][...] = acc[:, H:]
    else:
        outs[0][...] = acc


def _make_dense(relu, split):
    in_specs = [
        pl.BlockSpec((RB, H), lambda i: (i, 0)),
        pl.BlockSpec((RB, H), lambda i: (i, 0)),
        pl.BlockSpec((RB, 1), lambda i: (i, 0)),
        pl.BlockSpec((RB, H), lambda i: (i, 0)),
        pl.BlockSpec((RB, H), lambda i: (i, 0)),
        pl.BlockSpec((D, D), lambda i: (0, 0)),
        pl.BlockSpec((D, D), lambda i: (0, 0)),
        pl.BlockSpec((1, D), lambda i: (0, 0)),
    ]
    if split:
        out_specs = [pl.BlockSpec((RB, H), lambda i: (i, 0)),
                     pl.BlockSpec((RB, H), lambda i: (i, 0))]
        out_shape = [jax.ShapeDtypeStruct((N, H), jnp.float32),
                     jax.ShapeDtypeStruct((N, H), jnp.float32)]
    else:
        out_specs = pl.BlockSpec((RB, D), lambda i: (i, 0))
        out_shape = jax.ShapeDtypeStruct((N, D), jnp.float32)
    return pl.pallas_call(functools.partial(_dense_body, relu, split),
                          grid=(GRID,), in_specs=in_specs,
                          out_specs=out_specs, out_shape=out_shape)


_dense_mid = _make_dense(True, True)
_dense_fin = _make_dense(False, False)


def kernel(x, edge_index, W1_l, b1_l, W1_r, W2_l, b2_l, W2_r):
    src = edge_index[0].astype(jnp.int32)
    dst = edge_index[1].astype(jnp.int32)
    x_lo = x[:, :H]
    x_hi = x[:, H:]
    zrows = jnp.zeros((N, H), jnp.float32)
    zvec = jnp.zeros((N,), jnp.float32)

    s1_lo, s1_hi, cnt = _agg_cnt(x_lo, x_hi, src, dst, zrows, zvec)
    cnt2 = cnt.reshape(N, 1)
    h_lo, h_hi = _dense_mid(s1_lo, s1_hi, cnt2, x_lo, x_hi,
                            W1_l.T, W1_r.T, b1_l.reshape(1, D))
    s2_lo, s2_hi = _agg_nocnt(h_lo, h_hi, src, dst, zrows)
    out = _dense_fin(s2_lo, s2_hi, cnt2, h_lo, h_hi,
                     W2_l.T, W2_r.T, b2_l.reshape(1, D))
    return out
